# Initial kernel scaffold; baseline (speedup 1.0000x reference)
import jax
import jax.numpy as jnp
from jax.experimental import pallas as pl


def kernel(x, period, events, W_freq, b_freq, event_table, pe):
    raise NotImplementedError("write your pallas kernel here")


# speedup vs baseline: 3.7488x; 3.7488x over previous
import jax
import jax.numpy as jnp
from jax.experimental import pallas as pl

S_BLK = 256


def _pe_kernel(x_ref, pe_ref, ev_ref, period_ref, wf_ref, bf_ref, et_ref, o_ref):
    batch = x_ref.shape[1]
    pe = pe_ref[...]
    t0 = et_ref[0, :][None, :]
    delta = (et_ref[1, :] - et_ref[0, :])[None, :]
    for b in range(batch):
        combo_b = period_ref[0, b] * wf_ref[...] + bf_ref[...] + t0
        e_b = ev_ref[:, b][:, None]
        o_ref[:, b, :] = x_ref[:, b, :] + pe + combo_b + e_b * delta


def kernel(x, period, events, W_freq, b_freq, event_table, pe):
    seq_len, batch, d = x.shape
    ev_f = events.astype(jnp.float32)
    period2 = period.reshape(1, batch)
    wf2 = W_freq.reshape(1, d)
    bf2 = b_freq.reshape(1, d)

    grid = (seq_len // S_BLK,)
    return pl.pallas_call(
        _pe_kernel,
        grid=grid,
        in_specs=[
            pl.BlockSpec((S_BLK, batch, d), lambda i: (i, 0, 0)),
            pl.BlockSpec((S_BLK, d), lambda i: (i, 0)),
            pl.BlockSpec((S_BLK, batch), lambda i: (i, 0)),
            pl.BlockSpec((1, batch), lambda i: (0, 0)),
            pl.BlockSpec((1, d), lambda i: (0, 0)),
            pl.BlockSpec((1, d), lambda i: (0, 0)),
            pl.BlockSpec((2, d), lambda i: (0, 0)),
        ],
        out_specs=pl.BlockSpec((S_BLK, batch, d), lambda i: (i, 0, 0)),
        out_shape=jax.ShapeDtypeStruct((seq_len, batch, d), x.dtype),
    )(x, pe, ev_f, period2, wf2, bf2, event_table)


# speedup vs baseline: 3.9264x; 1.0474x over previous
import jax
import jax.numpy as jnp
from jax.experimental import pallas as pl

S_BLK = 512


def _pe_kernel(x_ref, pe_ref, ev_ref, period_ref, wf_ref, bf_ref, et_ref, o_ref):
    batch = x_ref.shape[1]
    pe = pe_ref[...]
    t0 = et_ref[0, :][None, :]
    delta = (et_ref[1, :] - et_ref[0, :])[None, :]
    for b in range(batch):
        combo_b = period_ref[0, b] * wf_ref[...] + bf_ref[...] + t0
        e_b = ev_ref[:, b][:, None]
        o_ref[:, b, :] = x_ref[:, b, :] + pe + combo_b + e_b * delta


def kernel(x, period, events, W_freq, b_freq, event_table, pe):
    seq_len, batch, d = x.shape
    ev_f = events.astype(jnp.float32)
    period2 = period.reshape(1, batch)
    wf2 = W_freq.reshape(1, d)
    bf2 = b_freq.reshape(1, d)

    grid = (seq_len // S_BLK,)
    return pl.pallas_call(
        _pe_kernel,
        grid=grid,
        in_specs=[
            pl.BlockSpec((S_BLK, batch, d), lambda i: (i, 0, 0)),
            pl.BlockSpec((S_BLK, d), lambda i: (i, 0)),
            pl.BlockSpec((S_BLK, batch), lambda i: (i, 0)),
            pl.BlockSpec((1, batch), lambda i: (0, 0)),
            pl.BlockSpec((1, d), lambda i: (0, 0)),
            pl.BlockSpec((1, d), lambda i: (0, 0)),
            pl.BlockSpec((2, d), lambda i: (0, 0)),
        ],
        out_specs=pl.BlockSpec((S_BLK, batch, d), lambda i: (i, 0, 0)),
        out_shape=jax.ShapeDtypeStruct((seq_len, batch, d), x.dtype),
    )(x, pe, ev_f, period2, wf2, bf2, event_table)
